# baseline (speedup 1.0000x reference)
import functools

import jax
import jax.numpy as jnp
from jax import lax
from jax.experimental import pallas as pl
from jax.experimental.pallas import tpu as pltpu
from jax.experimental.pallas import tpu_sc as plsc

_N = 16384 * 200
_V = 1000000
_NC = 2
_NS = 16
_NW = _NC * _NS
_BPW = _N // _NW
_C = 5120
_NCHUNKS = _BPW // _C
_L = 16
_UP = 8
_U2 = 8

_NPCHUNK = (_V + _C - 1) // _C
_SPW = (_NPCHUNK + _NS - 1) // _NS


def _sc_all(code, vals, mn_t, mx_t, mu_t, sd_t):
    mesh = plsc.VectorSubcoreMesh(core_axis_name="c", subcore_axis_name="s")

    @functools.partial(
        pl.kernel,
        mesh=mesh,
        out_type=jax.ShapeDtypeStruct((_N,), jnp.float32),
        compiler_params=pltpu.CompilerParams(
            needs_layout_passes=False, use_tc_tiling_on_sc=False),
        scratch_types=[
            pltpu.VMEM((_C,), jnp.int32),
            pltpu.VMEM((_C,), jnp.float32),
            pltpu.VMEM((_C,), jnp.uint32),
            pltpu.VMEM((_C,), jnp.float32),
            pltpu.VMEM((_C,), jnp.int32),
            pltpu.VMEM((_C,), jnp.float32),
            pltpu.VMEM((_C,), jnp.uint32),
            pltpu.VMEM((_C,), jnp.float32),
            pltpu.VMEM((_C,), jnp.float32),
            pltpu.VMEM((_C,), jnp.float32),
            pltpu.VMEM((_C,), jnp.float32),
            pltpu.VMEM((_C,), jnp.float32),
            pltpu.VMEM_SHARED((_V,), jnp.uint32),
            pltpu.SemaphoreType.DMA,
            pltpu.SemaphoreType.DMA,
            pltpu.SemaphoreType.DMA,
            pltpu.SemaphoreType.DMA,
            pltpu.SemaphoreType.DMA,
            pltpu.SemaphoreType.DMA,
        ],
    )
    def k(code_hbm, vals_hbm, mnt_hbm, mxt_hbm, mut_hbm, sdt_hbm, out_hbm,
          idx_a, vals_a, pk_a, out_a, idx_b, vals_b, pk_b, out_b,
          mu_a, sd_a, mu_b, sd_b, packed_sh,
          sem_ld_a, sem_ld_b, sem_a, sem_b, semo_a, semo_b):
        sid = lax.axis_index("s")
        wid = sid * _NC + lax.axis_index("c")
        base0 = wid * _BPW

        for g0 in (0, 1):
            gsem = (sem_a, sem_b)[g0 % 2]
            gidx = (idx_a, idx_b)[g0 % 2]
            gb = pl.multiple_of(base0 + g0 * _C, 8)
            pltpu.async_copy(code_hbm.at[pl.ds(gb, _C)], gidx, gsem)

        psets = ((vals_a, out_a, mu_a, sd_a, pk_a, sem_ld_a, semo_a),
                 (vals_b, out_b, mu_b, sd_b, pk_b, sem_ld_b, semo_b))

        def p_pred(jj):
            return sid + jj * _NS < _NPCHUNK

        def p_slice(jj):
            j = sid + jj * _NS
            sb = pl.multiple_of(jnp.minimum(j * _C, _V - _C), 8)
            return pl.ds(sb, _C)

        def p_copies(jj):
            mn_v, mx_v, mu_v, sd_v, pk_v, sem_ld, sem_st = psets[jj % 2]
            s = p_slice(jj)
            loads = ((mnt_hbm.at[s], mn_v), (mxt_hbm.at[s], mx_v),
                     (mut_hbm.at[s], mu_v), (sdt_hbm.at[s], sd_v))
            store = (pk_v, packed_sh.at[s])
            return loads, store, sem_ld, sem_st

        def fire_pload(jj):
            @pl.when(p_pred(jj))
            def _():
                loads, _, sem_ld, _ = p_copies(jj)
                for src, dst in loads:
                    pltpu.async_copy(src, dst, sem_ld)

        def wait_pload(jj):
            @pl.when(p_pred(jj))
            def _():
                loads, _, sem_ld, _ = p_copies(jj)
                for src, dst in loads:
                    pltpu.make_async_copy(src, dst, sem_ld).wait()

        def fire_pstore(jj):
            @pl.when(p_pred(jj))
            def _():
                _, (src, dst), _, sem_st = p_copies(jj)
                pltpu.async_copy(src, dst, sem_st)

        def wait_pstore(jj):
            @pl.when(p_pred(jj))
            def _():
                _, (src, dst), _, sem_st = p_copies(jj)
                pltpu.make_async_copy(src, dst, sem_st).wait()

        def compute_pack(jj):
            mn_v, mx_v, mu_v, sd_v, pk_v, _, _ = psets[jj % 2]

            @pl.when(p_pred(jj))
            def _():
                def grp(i0, c):
                    for u in range(_UP):
                        s = pl.ds((i0 * _UP + u) * _L, _L)
                        mn = mn_v[s]
                        pos = mn >= 0.0
                        off = jnp.where(pos, mn, mu_v[s])
                        den = jnp.where(pos, mx_v[s], sd_v[s])
                        scl = 1.0 / den
                        pair = plsc.pack(off, scl,
                                         format=plsc.PackFormat.INTERLEAVED)
                        pk_v[s] = plsc.bitcast(pair, jnp.uint32)
                    return c

                lax.fori_loop(0, _C // (_L * _UP), grp, 0)

        fire_pload(0)
        for jj in range(_SPW):
            if jj + 1 < _SPW:
                fire_pload(jj + 1)
            wait_pload(jj)
            if jj >= 2:
                wait_pstore(jj - 2)
            compute_pack(jj)
            fire_pstore(jj)
        for jj in (_SPW - 2, _SPW - 1):
            wait_pstore(jj)
        plsc.subcore_barrier()

        bufs = ((idx_a, vals_a, pk_a, out_a, sem_ld_a, sem_a, semo_a),
                (idx_b, vals_b, pk_b, out_b, sem_ld_b, sem_b, semo_b))

        def chunk_base(g):
            return pl.multiple_of(base0 + g * _C, 8)

        def fire_loads(g):
            idx_v, vals_v, _, _, sem_ld, _, _ = bufs[g % 2]
            base = chunk_base(g)
            pltpu.async_copy(code_hbm.at[pl.ds(base, _C)], idx_v, sem_ld)
            pltpu.async_copy(vals_hbm.at[pl.ds(base, _C)], vals_v, sem_ld)

        def fire_vals(g):
            _, vals_v, _, _, sem_ld, _, _ = bufs[g % 2]
            base = chunk_base(g)
            pltpu.async_copy(vals_hbm.at[pl.ds(base, _C)], vals_v, sem_ld)

        def wait_loads(g):
            idx_v, vals_v, _, _, sem_ld, sem, _ = bufs[g % 2]
            base = chunk_base(g)
            idx_sem = sem if g < 2 else sem_ld
            pltpu.make_async_copy(code_hbm.at[pl.ds(base, _C)], idx_v,
                                  idx_sem).wait()
            pltpu.make_async_copy(vals_hbm.at[pl.ds(base, _C)], vals_v,
                                  sem_ld).wait()

        def fire_gather(g):
            idx_v, _, pk_v, _, _, sem, _ = bufs[g % 2]
            pltpu.async_copy(packed_sh.at[idx_v], pk_v, sem)

        def wait_gather(g):
            idx_v, _, pk_v, _, _, sem, _ = bufs[g % 2]
            pltpu.make_async_copy(packed_sh.at[idx_v], pk_v, sem).wait()

        def compute(g):
            _, vals_v, pk_v, out_v, _, _, _ = bufs[g % 2]

            def grp(i0, c):
                for u in range(_U2):
                    s = pl.ds((i0 * _U2 + u) * _L, _L)
                    pair = plsc.bitcast(pk_v[s], jnp.bfloat16)
                    off, scl = plsc.unpack(pair,
                                           format=plsc.PackFormat.INTERLEAVED)
                    v = vals_v[s]
                    out_v[s] = ((v - off.astype(jnp.float32))
                                * scl.astype(jnp.float32))
                return c

            lax.fori_loop(0, _C // (_L * _U2), grp, 0)

        def fire_out(g):
            _, _, _, out_v, _, _, semo = bufs[g % 2]
            pltpu.async_copy(out_v, out_hbm.at[pl.ds(chunk_base(g), _C)], semo)

        def wait_out(g):
            _, _, _, out_v, _, _, semo = bufs[g % 2]
            pltpu.make_async_copy(out_v, out_hbm.at[pl.ds(chunk_base(g), _C)],
                                  semo).wait()

        fire_vals(0)
        fire_vals(1)
        wait_loads(0)
        fire_gather(0)
        for g in range(_NCHUNKS):
            if g + 1 < _NCHUNKS:
                wait_loads(g + 1)
                fire_gather(g + 1)
            wait_gather(g)
            if g >= 2:
                wait_out(g - 2)
            compute(g)
            fire_out(g)
            if g + 2 < _NCHUNKS:
                fire_loads(g + 2)
        wait_out(_NCHUNKS - 2)
        wait_out(_NCHUNKS - 1)

    return k(code, vals, mn_t, mx_t, mu_t, sd_t)


def kernel(values, code_index, min_val, max_val, mean, std):
    code = code_index.astype(jnp.int32)
    return _sc_all(code, values, min_val, max_val, mean, std)


# speedup vs baseline: 1.0139x; 1.0139x over previous
import functools

import jax
import jax.numpy as jnp
from jax import lax
from jax.experimental import pallas as pl
from jax.experimental.pallas import tpu as pltpu
from jax.experimental.pallas import tpu_sc as plsc

_N = 16384 * 200
_V = 1000000
_NC = 2
_NS = 16
_NW = _NC * _NS
_BPW = _N // _NW
_C = 5120
_NCHUNKS = _BPW // _C
_L = 16
_UP = 4
_U2 = 8

_NPCHUNK = (_V + _C - 1) // _C
_SPW = (_NPCHUNK + _NS - 1) // _NS


def _sc_all(code, vals, mn_t, mx_t, mu_t, sd_t):
    mesh = plsc.VectorSubcoreMesh(core_axis_name="c", subcore_axis_name="s")

    @functools.partial(
        pl.kernel,
        mesh=mesh,
        out_type=jax.ShapeDtypeStruct((_N,), jnp.float32),
        compiler_params=pltpu.CompilerParams(
            needs_layout_passes=False, use_tc_tiling_on_sc=False),
        scratch_types=[
            pltpu.VMEM((_C,), jnp.int32),
            pltpu.VMEM((_C,), jnp.float32),
            pltpu.VMEM((_C,), jnp.uint32),
            pltpu.VMEM((_C,), jnp.float32),
            pltpu.VMEM((_C,), jnp.int32),
            pltpu.VMEM((_C,), jnp.float32),
            pltpu.VMEM((_C,), jnp.uint32),
            pltpu.VMEM((_C,), jnp.float32),
            pltpu.VMEM((_C,), jnp.float32),
            pltpu.VMEM((_C,), jnp.float32),
            pltpu.VMEM((_C,), jnp.float32),
            pltpu.VMEM((_C,), jnp.float32),
            pltpu.VMEM_SHARED((_V,), jnp.uint32),
            pltpu.SemaphoreType.DMA,
            pltpu.SemaphoreType.DMA,
            pltpu.SemaphoreType.DMA,
            pltpu.SemaphoreType.DMA,
            pltpu.SemaphoreType.DMA,
            pltpu.SemaphoreType.DMA,
        ],
    )
    def k(code_hbm, vals_hbm, mnt_hbm, mxt_hbm, mut_hbm, sdt_hbm, out_hbm,
          idx_a, vals_a, pk_a, out_a, idx_b, vals_b, pk_b, out_b,
          mu_a, sd_a, mu_b, sd_b, packed_sh,
          sem_ld_a, sem_ld_b, sem_a, sem_b, semo_a, semo_b):
        sid = lax.axis_index("s")
        wid = sid * _NC + lax.axis_index("c")
        base0 = wid * _BPW

        for g0 in (0, 1):
            gsem = (sem_a, sem_b)[g0 % 2]
            gidx = (idx_a, idx_b)[g0 % 2]
            gb = pl.multiple_of(base0 + g0 * _C, 8)
            pltpu.async_copy(code_hbm.at[pl.ds(gb, _C)], gidx, gsem)

        psets = ((vals_a, out_a, mu_a, sd_a, pk_a, sem_ld_a, semo_a),
                 (vals_b, out_b, mu_b, sd_b, pk_b, sem_ld_b, semo_b))

        def p_pred(jj):
            return sid + jj * _NS < _NPCHUNK

        def p_slice(jj):
            j = sid + jj * _NS
            sb = pl.multiple_of(jnp.minimum(j * _C, _V - _C), 8)
            return pl.ds(sb, _C)

        def p_copies(jj):
            mn_v, mx_v, mu_v, sd_v, pk_v, sem_ld, sem_st = psets[jj % 2]
            s = p_slice(jj)
            loads = ((mnt_hbm.at[s], mn_v), (mxt_hbm.at[s], mx_v),
                     (mut_hbm.at[s], mu_v), (sdt_hbm.at[s], sd_v))
            store = (pk_v, packed_sh.at[s])
            return loads, store, sem_ld, sem_st

        def fire_pload(jj):
            @pl.when(p_pred(jj))
            def _():
                loads, _, sem_ld, _ = p_copies(jj)
                for src, dst in loads:
                    pltpu.async_copy(src, dst, sem_ld)

        def wait_pload(jj):
            @pl.when(p_pred(jj))
            def _():
                loads, _, sem_ld, _ = p_copies(jj)
                for src, dst in loads:
                    pltpu.make_async_copy(src, dst, sem_ld).wait()

        def fire_pstore(jj):
            @pl.when(p_pred(jj))
            def _():
                _, (src, dst), _, sem_st = p_copies(jj)
                pltpu.async_copy(src, dst, sem_st)

        def wait_pstore(jj):
            @pl.when(p_pred(jj))
            def _():
                _, (src, dst), _, sem_st = p_copies(jj)
                pltpu.make_async_copy(src, dst, sem_st).wait()

        def compute_pack(jj):
            mn_v, mx_v, mu_v, sd_v, pk_v, _, _ = psets[jj % 2]

            @pl.when(p_pred(jj))
            def _():
                def grp(i0, c):
                    for u in range(_UP):
                        s = pl.ds((i0 * _UP + u) * _L, _L)
                        mn = mn_v[s]
                        pos = mn >= 0.0
                        off = jnp.where(pos, mn, mu_v[s])
                        den = jnp.where(pos, mx_v[s], sd_v[s])
                        scl = 1.0 / den
                        pair = plsc.pack(off, scl,
                                         format=plsc.PackFormat.INTERLEAVED)
                        pk_v[s] = plsc.bitcast(pair, jnp.uint32)
                    return c

                lax.fori_loop(0, _C // (_L * _UP), grp, 0)

        fire_pload(0)
        for jj in range(_SPW):
            if jj + 1 < _SPW:
                fire_pload(jj + 1)
            wait_pload(jj)
            if jj >= 2:
                wait_pstore(jj - 2)
            compute_pack(jj)
            fire_pstore(jj)
        for jj in (_SPW - 2, _SPW - 1):
            wait_pstore(jj)
        plsc.subcore_barrier()

        bufs = ((idx_a, vals_a, pk_a, out_a, sem_ld_a, sem_a, semo_a),
                (idx_b, vals_b, pk_b, out_b, sem_ld_b, sem_b, semo_b))

        def chunk_base(g):
            return pl.multiple_of(base0 + g * _C, 8)

        def fire_loads(g):
            idx_v, vals_v, _, _, sem_ld, _, _ = bufs[g % 2]
            base = chunk_base(g)
            pltpu.async_copy(code_hbm.at[pl.ds(base, _C)], idx_v, sem_ld)
            pltpu.async_copy(vals_hbm.at[pl.ds(base, _C)], vals_v, sem_ld)

        def fire_vals(g):
            _, vals_v, _, _, sem_ld, _, _ = bufs[g % 2]
            base = chunk_base(g)
            pltpu.async_copy(vals_hbm.at[pl.ds(base, _C)], vals_v, sem_ld)

        def wait_loads(g):
            idx_v, vals_v, _, _, sem_ld, sem, _ = bufs[g % 2]
            base = chunk_base(g)
            idx_sem = sem if g < 2 else sem_ld
            pltpu.make_async_copy(code_hbm.at[pl.ds(base, _C)], idx_v,
                                  idx_sem).wait()
            pltpu.make_async_copy(vals_hbm.at[pl.ds(base, _C)], vals_v,
                                  sem_ld).wait()

        def fire_gather(g):
            idx_v, _, pk_v, _, _, sem, _ = bufs[g % 2]
            pltpu.async_copy(packed_sh.at[idx_v], pk_v, sem)

        def wait_gather(g):
            idx_v, _, pk_v, _, _, sem, _ = bufs[g % 2]
            pltpu.make_async_copy(packed_sh.at[idx_v], pk_v, sem).wait()

        def compute(g):
            _, vals_v, pk_v, out_v, _, _, _ = bufs[g % 2]

            def grp(i0, c):
                for u in range(_U2):
                    s = pl.ds((i0 * _U2 + u) * _L, _L)
                    pair = plsc.bitcast(pk_v[s], jnp.bfloat16)
                    off, scl = plsc.unpack(pair,
                                           format=plsc.PackFormat.INTERLEAVED)
                    v = vals_v[s]
                    out_v[s] = ((v - off.astype(jnp.float32))
                                * scl.astype(jnp.float32))
                return c

            lax.fori_loop(0, _C // (_L * _U2), grp, 0)

        def fire_out(g):
            _, _, _, out_v, _, _, semo = bufs[g % 2]
            pltpu.async_copy(out_v, out_hbm.at[pl.ds(chunk_base(g), _C)], semo)

        def wait_out(g):
            _, _, _, out_v, _, _, semo = bufs[g % 2]
            pltpu.make_async_copy(out_v, out_hbm.at[pl.ds(chunk_base(g), _C)],
                                  semo).wait()

        fire_vals(0)
        fire_vals(1)
        wait_loads(0)
        fire_gather(0)
        for g in range(_NCHUNKS):
            if g + 1 < _NCHUNKS:
                wait_loads(g + 1)
                fire_gather(g + 1)
            wait_gather(g)
            if g >= 2:
                wait_out(g - 2)
            compute(g)
            fire_out(g)
            if g + 2 < _NCHUNKS:
                fire_loads(g + 2)
        wait_out(_NCHUNKS - 2)
        wait_out(_NCHUNKS - 1)

    return k(code, vals, mn_t, mx_t, mu_t, sd_t)


def kernel(values, code_index, min_val, max_val, mean, std):
    code = code_index.astype(jnp.int32)
    return _sc_all(code, values, min_val, max_val, mean, std)
